# baseline (speedup 1.0000x reference)
import math

import jax
import jax.numpy as jnp
from jax import lax
from jax.experimental import pallas as pl
from jax.experimental.pallas import tpu as pltpu
from jax.experimental.pallas import tpu_sc as plsc

B, N, DIM, MAXB, HID = 8, 5000, 512, 3, 16
NPAD = 5120
NW = 1280
PW = 128
DH = DIM // 2
NPANEL = NW // PW
NTAIL = N - 39 * 128
NALIGN = N - NTAIL
NEG = -1e30


def _sc_score_kernel(text_hbm, x_hbm, dots_hbm, ssqs_hbm,
                     tf_all, bufA, bufB, od, oq, semA, semB):
    c = lax.axis_index("c")
    s = lax.axis_index("s")
    w = c * 16 + s
    b = w // 4
    q = w - b * 4
    n0 = jnp.where(q < 3, q * NW, NALIGN - NW)

    pltpu.sync_copy(text_hbm, tf_all)

    def start(p, h, buf, sem):
        pltpu.make_async_copy(
            x_hbm.at[b, 0, pl.ds(h * DH, DH), pl.ds(n0 + p * PW, PW)],
            buf, sem).start()

    def wait(buf, sem):
        pltpu.make_async_copy(
            x_hbm.at[b, 0, pl.ds(0, DH), pl.ds(n0, PW)], buf, sem).wait()

    def compute(p, h, buf):
        for j in range(PW // 16):
            def dbody(dc, carry):
                a1, a2 = carry
                tfv = tf_all[b, pl.ds(h * DH + dc * 16, 16)]
                for i in range(16):
                    v = buf[dc * 16 + i, pl.ds(j * 16, 16)]
                    t = tfv[i]
                    a1 = a1 + v * t
                    a2 = a2 + v * v
                return (a1, a2)
            a1, a2 = lax.fori_loop(
                0, DH // 16, dbody,
                (jnp.zeros((16,), jnp.float32),
                 jnp.zeros((16,), jnp.float32)))
            sl = pl.ds(p * PW + j * 16, 16)
            if h == 0:
                od[sl] = a1
                oq[sl] = a2
            else:
                od[sl] = od[sl] + a1
                oq[sl] = oq[sl] + a2

    start(0, 0, bufA, semA)

    def outer(i, carry):
        start(i, 1, bufB, semB)
        wait(bufA, semA)
        compute(i, 0, bufA)

        @pl.when(i < NPANEL - 1)
        def _():
            start(i + 1, 0, bufA, semA)

        wait(bufB, semB)
        compute(i, 1, bufB)
        return 0

    lax.fori_loop(0, NPANEL, outer, 0)
    pltpu.sync_copy(od, dots_hbm.at[b, pl.ds(n0, NW)])
    pltpu.sync_copy(oq, ssqs_hbm.at[b, pl.ds(n0, NW)])


def _tc_epilogue_kernel(text_ref, dots_ref, ssqs_ref, x_ref, boxes_ref,
                        Wq_ref, bq_ref, Wk_ref, bk_ref, Wv_ref, bv_ref,
                        Wo_ref, bo_ref, W1_ref, b1_ref, g1_ref, be1_ref,
                        W2_ref, b2_ref, out_ref, colbuf, tailbuf, dsem):
    tf = text_ref[...]
    tnorm = jnp.sqrt(jnp.sum(tf * tf, axis=1, keepdims=True)) + 1e-8
    lanes = jax.lax.broadcasted_iota(jnp.int32, (B, NPAD), 1)
    score = ((100.0 * dots_ref[...])
             / ((jnp.sqrt(ssqs_ref[...]) + 1e-8) * tnorm))
    S = jnp.where(lanes < NALIGN, score, NEG)

    tail_cps = [pltpu.make_async_copy(
        x_ref.at[bb, 0, :, pl.ds(NALIGN, NTAIL)], tailbuf.at[bb], dsem)
        for bb in range(B)]
    for cp in tail_cps:
        cp.start()
    for cp in tail_cps:
        cp.wait()
    tail_rows = []
    for bb in range(B):
        win = tailbuf[bb]
        d8 = jnp.dot(tf[bb:bb + 1, :], win,
                     preferred_element_type=jnp.float32)
        s8 = jnp.sum(win * win, axis=0, keepdims=True)
        tail_rows.append((100.0 * d8)
                         / ((jnp.sqrt(s8) + 1e-8) * tnorm[bb:bb + 1, :]))
    tail8 = jnp.concatenate(tail_rows, axis=0)
    place = (lanes[:NTAIL, :] == (NALIGN + jax.lax.broadcasted_iota(
        jnp.int32, (NTAIL, NPAD), 0))).astype(jnp.float32)
    tail_big = jnp.dot(tail8, place, preferred_element_type=jnp.float32)
    S = jnp.where((lanes >= NALIGN) & (lanes < N), tail_big, S)

    v_cols, n_cols = [], []
    for _ in range(MAXB):
        m = jnp.max(S, axis=1, keepdims=True)
        p = jnp.min(jnp.where(S == m, lanes, NPAD), axis=1,
                    keepdims=True)
        v_cols.append(m)
        n_cols.append(p)
        S = jnp.where(lanes == p, NEG, S)
    v_b3 = jnp.concatenate(v_cols, axis=1)
    n_b3 = jnp.concatenate(n_cols, axis=1)

    rowB = jax.lax.broadcasted_iota(jnp.int32, (B, MAXB), 0)
    col3 = jax.lax.broadcasted_iota(jnp.int32, (B, MAXB), 1)
    copies, offs = [], []
    for m in range(MAXB):
        for bb in range(B):
            nsc = jnp.sum(jnp.where((rowB == bb) & (col3 == m), n_b3, 0))
            base = pl.multiple_of((nsc // 128) * 128, 128)
            cp = pltpu.make_async_copy(
                x_ref.at[bb, 0, :, pl.ds(base, 128)],
                colbuf.at[m * B + bb], dsem)
            cp.start()
            copies.append(cp)
            offs.append(nsc - base)
    for cp in copies:
        cp.wait()
    lane128 = jax.lax.broadcasted_iota(jnp.int32, (1, 128), 1)
    H = jnp.concatenate(
        [lax.dot_general((lane128 == offs[j]).astype(jnp.float32),
                         colbuf[j], (((1,), (1,)), ((), ())),
                         preferred_element_type=jnp.float32)
         for j in range(MAXB * B)], axis=0)

    qc = (jnp.dot(H, Wq_ref[...],
                  preferred_element_type=jnp.float32) + bq_ref[...])
    kc = (jnp.dot(H, Wk_ref[...],
                  preferred_element_type=jnp.float32) + bk_ref[...])
    vc = (jnp.dot(H, Wv_ref[...],
                  preferred_element_type=jnp.float32) + bv_ref[...])
    wo = (jnp.dot(vc, Wo_ref[...], preferred_element_type=jnp.float32)
          + bo_ref[...])
    qm = [qc[B * m:B * (m + 1), :] for m in range(MAXB)]
    km = [kc[B * m:B * (m + 1), :] for m in range(MAXB)]
    wom = [wo[B * m:B * (m + 1), :] for m in range(MAXB)]
    inv_sqrt_d = 1.0 / math.sqrt(float(DIM))
    gam_cols = []
    for m in range(MAXB):
        L = jnp.concatenate(
            [jnp.sum(qm[m] * km[mp], axis=1, keepdims=True) * inv_sqrt_d
             for mp in range(MAXB)], axis=1)
        L = L - jnp.max(L, axis=1, keepdims=True)
        e = jnp.exp(L)
        attn = e / jnp.sum(e, axis=1, keepdims=True)
        gam_cols.append(sum(attn[:, mp:mp + 1] * wom[mp]
                            for mp in range(MAXB)))
    xs = jnp.concatenate(gam_cols, axis=1) + v_b3

    tvec = jnp.max(xs, axis=1, keepdims=True)
    jstar = jnp.min(jnp.where(xs == tvec, col3, MAXB), axis=1,
                    keepdims=True)
    n_sel = jnp.sum(jnp.where(col3 == jstar, n_b3, 0), axis=1,
                    keepdims=True)

    r1 = tvec * W1_ref[...] + b1_ref[...]
    r1 = 0.5 * r1 * (1.0 + jax.lax.erf(r1 / math.sqrt(2.0)))
    mu = jnp.mean(r1, axis=1, keepdims=True)
    var = jnp.mean((r1 - mu) ** 2, axis=1, keepdims=True)
    r1 = (r1 - mu) / jnp.sqrt(var + 1e-5) * g1_ref[...] + be1_ref[...]
    r2 = jnp.dot(r1, W2_ref[...], preferred_element_type=jnp.float32)
    r2 = jnp.maximum(r2 + b2_ref[...], 0.0)

    boxes_sel = []
    for bb in range(B):
        nb = jnp.sum(jnp.where(
            jax.lax.broadcasted_iota(jnp.int32, (B, 1), 0) == bb, n_sel, 0))
        boxes_sel.append(boxes_ref[bb, pl.ds(nb, 1), :])
    out_ref[...] = r2 + jnp.concatenate(boxes_sel, axis=0)


@jax.jit
def kernel(text_feat, x, boxes, Wq, bq, Wk, bk, Wv, bv, Wo, bo,
           W1, b1, g1, be1, W2, b2):
    sc_score = pl.kernel(
        _sc_score_kernel,
        mesh=plsc.VectorSubcoreMesh(core_axis_name="c", subcore_axis_name="s"),
        out_type=[
            jax.ShapeDtypeStruct((B, NPAD), jnp.float32),
            jax.ShapeDtypeStruct((B, NPAD), jnp.float32),
        ],
        scratch_types=[
            pltpu.VMEM((B, DIM), jnp.float32),
            pltpu.VMEM((DH, PW), jnp.float32),
            pltpu.VMEM((DH, PW), jnp.float32),
            pltpu.VMEM((NW,), jnp.float32),
            pltpu.VMEM((NW,), jnp.float32),
            pltpu.SemaphoreType.DMA,
            pltpu.SemaphoreType.DMA,
        ],
    )
    dots, ssqs = sc_score(text_feat, x)

    out = pl.pallas_call(
        _tc_epilogue_kernel,
        in_specs=[
            pl.BlockSpec((B, DIM), lambda: (0, 0)),
            pl.BlockSpec((B, NPAD), lambda: (0, 0)),
            pl.BlockSpec((B, NPAD), lambda: (0, 0)),
            pl.BlockSpec(memory_space=pl.ANY),
            pl.BlockSpec((B, N, 4), lambda: (0, 0, 0)),
            pl.BlockSpec((DIM, DIM), lambda: (0, 0)),
            pl.BlockSpec((1, DIM), lambda: (0, 0)),
            pl.BlockSpec((DIM, DIM), lambda: (0, 0)),
            pl.BlockSpec((1, DIM), lambda: (0, 0)),
            pl.BlockSpec((DIM, DIM), lambda: (0, 0)),
            pl.BlockSpec((1, DIM), lambda: (0, 0)),
            pl.BlockSpec((DIM, 1), lambda: (0, 0)),
            pl.BlockSpec((1, 1), lambda: (0, 0)),
            pl.BlockSpec((1, HID), lambda: (0, 0)),
            pl.BlockSpec((1, HID), lambda: (0, 0)),
            pl.BlockSpec((1, HID), lambda: (0, 0)),
            pl.BlockSpec((1, HID), lambda: (0, 0)),
            pl.BlockSpec((HID, 4), lambda: (0, 0)),
            pl.BlockSpec((1, 4), lambda: (0, 0)),
        ],
        out_specs=pl.BlockSpec((B, 4), lambda: (0, 0)),
        out_shape=jax.ShapeDtypeStruct((B, 4), jnp.float32),
        scratch_shapes=[
            pltpu.VMEM((MAXB * B, DIM, 128), jnp.float32),
            pltpu.VMEM((B, DIM, NTAIL), jnp.float32),
            pltpu.SemaphoreType.DMA,
        ],
    )(text_feat, dots, ssqs, x, boxes, Wq, bq.reshape(1, DIM),
      Wk, bk.reshape(1, DIM), Wv, bv.reshape(1, DIM), Wo, bo.reshape(1, 1),
      W1, b1.reshape(1, HID), g1.reshape(1, HID), be1.reshape(1, HID),
      W2, b2.reshape(1, 4))
    return out


# speedup vs baseline: 1.1515x; 1.1515x over previous
import math

import jax
import jax.numpy as jnp
from jax import lax
from jax.experimental import pallas as pl
from jax.experimental.pallas import tpu as pltpu
from jax.experimental.pallas import tpu_sc as plsc

B, N, DIM, MAXB, HID = 8, 5000, 512, 3, 16
BT = 5
BS = B - BT
NPAD = 5120
NBLK = 2560
NB = 2
NWS = 512
PW = 128
DH = DIM // 2
NPANEL = NWS // PW
NTAIL = N - 39 * 128
NALIGN = N - NTAIL
NEG = -1e30


def _tc_score_kernel(text_ref, x_ref, cv_ref, ci_ref, cc_ref):
    k = pl.program_id(1)
    tf = text_ref[pl.ds(pl.program_id(0), 1), :]
    col = jax.lax.broadcasted_iota(jnp.int32, (1, NBLK), 1)
    n0 = k * NBLK
    valid = (n0 + col) < N
    feat = x_ref[0, 0]

    dot = jnp.dot(tf, feat, preferred_element_type=jnp.float32)
    ssq = jnp.sum(feat * feat, axis=0, keepdims=True)
    tnorm = jnp.sqrt(jnp.sum(tf * tf)) + 1e-8
    score = (100.0 * dot) / ((jnp.sqrt(ssq) + 1e-8) * tnorm)
    score = jnp.where(valid, score, NEG)

    vals, idxs = [], []
    cur = score
    for _ in range(MAXB):
        m = jnp.max(cur)
        i = jnp.min(jnp.where(cur == m, col, NBLK))
        vals.append(m)
        idxs.append(i)
        cur = jnp.where(col == i, NEG, cur)

    row3 = jax.lax.broadcasted_iota(jnp.int32, (MAXB, 1), 0)
    idx_mat = (idxs[0] * (row3 == 0) + idxs[1] * (row3 == 1)
               + idxs[2] * (row3 == 2))
    oh = (jax.lax.broadcasted_iota(jnp.int32, (MAXB, NBLK), 1)
          == idx_mat).astype(jnp.float32)

    @pl.when(k < NB - 1)
    def _():
        cc_ref[0, 0] = jax.lax.dot_general(
            oh, feat, (((1,), (1,)), ((), ())),
            preferred_element_type=jnp.float32)

    @pl.when(k == NB - 1)
    def _():
        featm = jnp.where(valid, feat, 0.0)
        cc_ref[0, 0] = jax.lax.dot_general(
            oh, featm, (((1,), (1,)), ((), ())),
            preferred_element_type=jnp.float32)

    lane = jax.lax.broadcasted_iota(jnp.int32, (1, 128), 1)
    vvec = jnp.full((1, 128), NEG, jnp.float32)
    ivec = jnp.zeros((1, 128), jnp.int32)
    for j in range(MAXB):
        vvec = jnp.where(lane == j, vals[j], vvec)
        ivec = jnp.where(lane == j, idxs[j] + n0, ivec)
    cv_ref[0, 0] = vvec
    ci_ref[0, 0] = ivec


def _sc_score_kernel(text_hbm, x_hbm, dots_hbm, ssqs_hbm,
                     tf_all, bufA, bufB, od, oq, semA, semB):
    c = lax.axis_index("c")
    s = lax.axis_index("s")
    w = c * 16 + s

    @pl.when(w < BS * 10)
    def _():
        b = BT + w // 10
        q = w - (w // 10) * 10
        n0 = jnp.where(q < 9, q * NWS, NALIGN - NWS)

        pltpu.sync_copy(text_hbm, tf_all)

        def start(p, h, buf, sem):
            pltpu.make_async_copy(
                x_hbm.at[b, 0, pl.ds(h * DH, DH), pl.ds(n0 + p * PW, PW)],
                buf, sem).start()

        def wait(buf, sem):
            pltpu.make_async_copy(
                x_hbm.at[b, 0, pl.ds(0, DH), pl.ds(n0, PW)], buf, sem).wait()

        def compute(p, h, buf):
            for j in range(PW // 16):
                def dbody(dc, carry):
                    a1, a2 = carry
                    tfv = tf_all[b, pl.ds(h * DH + dc * 16, 16)]
                    for i in range(16):
                        v = buf[dc * 16 + i, pl.ds(j * 16, 16)]
                        t = tfv[i]
                        a1 = a1 + v * t
                        a2 = a2 + v * v
                    return (a1, a2)
                a1, a2 = lax.fori_loop(
                    0, DH // 16, dbody,
                    (jnp.zeros((16,), jnp.float32),
                     jnp.zeros((16,), jnp.float32)))
                sl = pl.ds(p * PW + j * 16, 16)
                if h == 0:
                    od[sl] = a1
                    oq[sl] = a2
                else:
                    od[sl] = od[sl] + a1
                    oq[sl] = oq[sl] + a2

        start(0, 0, bufA, semA)

        def outer(i, carry):
            start(i, 1, bufB, semB)
            wait(bufA, semA)
            compute(i, 0, bufA)

            @pl.when(i < NPANEL - 1)
            def _():
                start(i + 1, 0, bufA, semA)

            wait(bufB, semB)
            compute(i, 1, bufB)
            return 0

        lax.fori_loop(0, NPANEL, outer, 0)
        pltpu.sync_copy(od, dots_hbm.at[b, pl.ds(n0, NWS)])
        pltpu.sync_copy(oq, ssqs_hbm.at[b, pl.ds(n0, NWS)])


def _tc_epilogue_kernel(text_ref, cv_ref, ci_ref, cc_ref,
                        dots_ref, ssqs_ref, x_ref, boxes_ref,
                        Wq_ref, bq_ref, Wk_ref, bk_ref, Wv_ref, bv_ref,
                        Wo_ref, bo_ref, W1_ref, b1_ref, g1_ref, be1_ref,
                        W2_ref, b2_ref, out_ref, colbuf, tailbuf, dsem):
    tf = text_ref[...]
    tnorm = jnp.sqrt(jnp.sum(tf * tf, axis=1, keepdims=True)) + 1e-8

    Stc = jnp.concatenate(
        [jnp.concatenate([cv_ref[bb].reshape(NB, 128)[kk:kk + 1, :]
                          for kk in range(NB)], axis=1)
         for bb in range(BT)], axis=0)
    Itc = jnp.concatenate(
        [jnp.concatenate([ci_ref[bb].reshape(NB, 128)[kk:kk + 1, :]
                          for kk in range(NB)], axis=1)
         for bb in range(BT)], axis=0)
    pos_t = jax.lax.broadcasted_iota(jnp.int32, (BT, NB * 128), 1)
    vt_cols, nt_cols, pt_cols = [], [], []
    Swork = Stc
    for _ in range(MAXB):
        m = jnp.max(Swork, axis=1, keepdims=True)
        p = jnp.min(jnp.where(Swork == m, pos_t, NB * 128), axis=1,
                    keepdims=True)
        n_orig = jnp.sum(jnp.where(pos_t == p, Itc, 0), axis=1,
                         keepdims=True)
        vt_cols.append(m)
        pt_cols.append(p)
        nt_cols.append(n_orig)
        Swork = jnp.where(pos_t == p, NEG, Swork)
    vt_b3 = jnp.concatenate(vt_cols, axis=1)
    nt_b3 = jnp.concatenate(nt_cols, axis=1)
    pt_b3 = jnp.concatenate(pt_cols, axis=1)
    rt = pt_b3 // 128
    ct = pt_b3 - rt * 128
    rowT = jax.lax.broadcasted_iota(jnp.int32, (BT, MAXB), 0)
    rc_t = (rowT * NB + rt) * MAXB + ct

    C_all = jnp.concatenate(
        [cc_ref[bb].reshape(NB * MAXB, DIM) for bb in range(BT)],
        axis=0)
    laneC = jax.lax.broadcasted_iota(jnp.int32, (BT, BT * NB * MAXB), 1)
    Htc = [jnp.dot((laneC == rc_t[:, m:m + 1]).astype(jnp.float32), C_all,
                   preferred_element_type=jnp.float32)
           for m in range(MAXB)]

    lanes = jax.lax.broadcasted_iota(jnp.int32, (BS, NPAD), 1)
    tn_s = tnorm[BT:, :]
    score_s = ((100.0 * dots_ref[BT:, :])
               / ((jnp.sqrt(ssqs_ref[BT:, :]) + 1e-8) * tn_s))
    Ssc = jnp.where(lanes < NALIGN, score_s, NEG)

    tail_cps = [pltpu.make_async_copy(
        x_ref.at[BT + bb, 0, :, pl.ds(NALIGN, NTAIL)], tailbuf.at[bb], dsem)
        for bb in range(BS)]
    for cp in tail_cps:
        cp.start()
    for cp in tail_cps:
        cp.wait()
    tail_rows = []
    for bb in range(BS):
        win = tailbuf[bb]
        d8 = jnp.dot(tf[BT + bb:BT + bb + 1, :], win,
                     preferred_element_type=jnp.float32)
        s8 = jnp.sum(win * win, axis=0, keepdims=True)
        tail_rows.append((100.0 * d8)
                         / ((jnp.sqrt(s8) + 1e-8) * tn_s[bb:bb + 1, :]))
    tail8 = jnp.concatenate(tail_rows, axis=0)
    place = (jax.lax.broadcasted_iota(jnp.int32, (NTAIL, NPAD), 1)
             == (NALIGN + jax.lax.broadcasted_iota(
                 jnp.int32, (NTAIL, NPAD), 0))).astype(jnp.float32)
    tail_big = jnp.dot(tail8, place, preferred_element_type=jnp.float32)
    Ssc = jnp.where((lanes >= NALIGN) & (lanes < N), tail_big, Ssc)

    vs_cols, ns_cols = [], []
    for _ in range(MAXB):
        m = jnp.max(Ssc, axis=1, keepdims=True)
        p = jnp.min(jnp.where(Ssc == m, lanes, NPAD), axis=1, keepdims=True)
        vs_cols.append(m)
        ns_cols.append(p)
        Ssc = jnp.where(lanes == p, NEG, Ssc)
    vs_b3 = jnp.concatenate(vs_cols, axis=1)
    ns_b3 = jnp.concatenate(ns_cols, axis=1)

    rowS = jax.lax.broadcasted_iota(jnp.int32, (BS, MAXB), 0)
    colS = jax.lax.broadcasted_iota(jnp.int32, (BS, MAXB), 1)
    copies, offs = [], []
    for m in range(MAXB):
        for bb in range(BS):
            nsc = jnp.sum(jnp.where((rowS == bb) & (colS == m), ns_b3, 0))
            base = pl.multiple_of((nsc // 128) * 128, 128)
            cp = pltpu.make_async_copy(
                x_ref.at[BT + bb, 0, :, pl.ds(base, 128)],
                colbuf.at[m * BS + bb], dsem)
            cp.start()
            copies.append(cp)
            offs.append(nsc - base)
    for cp in copies:
        cp.wait()
    lane128 = jax.lax.broadcasted_iota(jnp.int32, (1, 128), 1)
    Hsc = [jnp.concatenate(
        [lax.dot_general((lane128 == offs[m * BS + bb]).astype(jnp.float32),
                         colbuf[m * BS + bb], (((1,), (1,)), ((), ())),
                         preferred_element_type=jnp.float32)
         for bb in range(BS)], axis=0) for m in range(MAXB)]

    v_b3 = jnp.concatenate([vt_b3, vs_b3], axis=0)
    n_b3 = jnp.concatenate([nt_b3, ns_b3], axis=0)
    H = jnp.concatenate([jnp.concatenate([Htc[m], Hsc[m]], axis=0)
                         for m in range(MAXB)], axis=0)

    qc = (jnp.dot(H, Wq_ref[...],
                  preferred_element_type=jnp.float32) + bq_ref[...])
    kc = (jnp.dot(H, Wk_ref[...],
                  preferred_element_type=jnp.float32) + bk_ref[...])
    vc = (jnp.dot(H, Wv_ref[...],
                  preferred_element_type=jnp.float32) + bv_ref[...])
    wo = (jnp.dot(vc, Wo_ref[...], preferred_element_type=jnp.float32)
          + bo_ref[...])
    qm = [qc[B * m:B * (m + 1), :] for m in range(MAXB)]
    km = [kc[B * m:B * (m + 1), :] for m in range(MAXB)]
    wom = [wo[B * m:B * (m + 1), :] for m in range(MAXB)]
    inv_sqrt_d = 1.0 / math.sqrt(float(DIM))
    gam_cols = []
    for m in range(MAXB):
        L = jnp.concatenate(
            [jnp.sum(qm[m] * km[mp], axis=1, keepdims=True) * inv_sqrt_d
             for mp in range(MAXB)], axis=1)
        L = L - jnp.max(L, axis=1, keepdims=True)
        e = jnp.exp(L)
        attn = e / jnp.sum(e, axis=1, keepdims=True)
        gam_cols.append(sum(attn[:, mp:mp + 1] * wom[mp]
                            for mp in range(MAXB)))
    xs = jnp.concatenate(gam_cols, axis=1) + v_b3

    col3 = jax.lax.broadcasted_iota(jnp.int32, (B, MAXB), 1)
    tvec = jnp.max(xs, axis=1, keepdims=True)
    jstar = jnp.min(jnp.where(xs == tvec, col3, MAXB), axis=1,
                    keepdims=True)
    n_sel = jnp.sum(jnp.where(col3 == jstar, n_b3, 0), axis=1,
                    keepdims=True)

    r1 = tvec * W1_ref[...] + b1_ref[...]
    r1 = 0.5 * r1 * (1.0 + jax.lax.erf(r1 / math.sqrt(2.0)))
    mu = jnp.mean(r1, axis=1, keepdims=True)
    var = jnp.mean((r1 - mu) ** 2, axis=1, keepdims=True)
    r1 = (r1 - mu) / jnp.sqrt(var + 1e-5) * g1_ref[...] + be1_ref[...]
    r2 = jnp.dot(r1, W2_ref[...], preferred_element_type=jnp.float32)
    r2 = jnp.maximum(r2 + b2_ref[...], 0.0)

    boxes_sel = []
    for bb in range(B):
        nb = jnp.sum(jnp.where(
            jax.lax.broadcasted_iota(jnp.int32, (B, 1), 0) == bb, n_sel, 0))
        boxes_sel.append(boxes_ref[bb, pl.ds(nb, 1), :])
    out_ref[...] = r2 + jnp.concatenate(boxes_sel, axis=0)


@jax.jit
def kernel(text_feat, x, boxes, Wq, bq, Wk, bk, Wv, bv, Wo, bo,
           W1, b1, g1, be1, W2, b2):
    sc_score = pl.kernel(
        _sc_score_kernel,
        mesh=plsc.VectorSubcoreMesh(core_axis_name="c", subcore_axis_name="s"),
        out_type=[
            jax.ShapeDtypeStruct((B, NPAD), jnp.float32),
            jax.ShapeDtypeStruct((B, NPAD), jnp.float32),
        ],
        scratch_types=[
            pltpu.VMEM((B, DIM), jnp.float32),
            pltpu.VMEM((DH, PW), jnp.float32),
            pltpu.VMEM((DH, PW), jnp.float32),
            pltpu.VMEM((NWS,), jnp.float32),
            pltpu.VMEM((NWS,), jnp.float32),
            pltpu.SemaphoreType.DMA,
            pltpu.SemaphoreType.DMA,
        ],
    )
    dots, ssqs = sc_score(text_feat, x)

    cv, ci, cc = pl.pallas_call(
        _tc_score_kernel,
        grid=(BT, NB),
        in_specs=[
            pl.BlockSpec((B, DIM), lambda b, k: (0, 0)),
            pl.BlockSpec((1, 1, DIM, NBLK), lambda b, k: (b, 0, 0, k)),
        ],
        out_specs=[
            pl.BlockSpec((1, 1, 1, 128), lambda b, k: (b, k, 0, 0)),
            pl.BlockSpec((1, 1, 1, 128), lambda b, k: (b, k, 0, 0)),
            pl.BlockSpec((1, 1, MAXB, DIM), lambda b, k: (b, k, 0, 0)),
        ],
        out_shape=[
            jax.ShapeDtypeStruct((BT, NB, 1, 128), jnp.float32),
            jax.ShapeDtypeStruct((BT, NB, 1, 128), jnp.int32),
            jax.ShapeDtypeStruct((BT, NB, MAXB, DIM), jnp.float32),
        ],
    )(text_feat, x)

    rep2 = lambda: (0, 0)
    out = pl.pallas_call(
        _tc_epilogue_kernel,
        in_specs=[
            pl.BlockSpec((B, DIM), rep2),
            pl.BlockSpec((BT, NB, 1, 128), lambda: (0, 0, 0, 0)),
            pl.BlockSpec((BT, NB, 1, 128), lambda: (0, 0, 0, 0)),
            pl.BlockSpec((BT, NB, MAXB, DIM), lambda: (0, 0, 0, 0)),
            pl.BlockSpec((B, NPAD), rep2),
            pl.BlockSpec((B, NPAD), rep2),
            pl.BlockSpec(memory_space=pl.ANY),
            pl.BlockSpec((B, N, 4), lambda: (0, 0, 0)),
            pl.BlockSpec((DIM, DIM), rep2),
            pl.BlockSpec((1, DIM), rep2),
            pl.BlockSpec((DIM, DIM), rep2),
            pl.BlockSpec((1, DIM), rep2),
            pl.BlockSpec((DIM, DIM), rep2),
            pl.BlockSpec((1, DIM), rep2),
            pl.BlockSpec((DIM, 1), rep2),
            pl.BlockSpec((1, 1), rep2),
            pl.BlockSpec((1, HID), rep2),
            pl.BlockSpec((1, HID), rep2),
            pl.BlockSpec((1, HID), rep2),
            pl.BlockSpec((1, HID), rep2),
            pl.BlockSpec((HID, 4), rep2),
            pl.BlockSpec((1, 4), rep2),
        ],
        out_specs=pl.BlockSpec((B, 4), rep2),
        out_shape=jax.ShapeDtypeStruct((B, 4), jnp.float32),
        scratch_shapes=[
            pltpu.VMEM((MAXB * BS, DIM, 128), jnp.float32),
            pltpu.VMEM((BS, DIM, NTAIL), jnp.float32),
            pltpu.SemaphoreType.DMA,
        ],
    )(text_feat, cv, ci, cc, dots, ssqs, x, boxes, Wq, bq.reshape(1, DIM),
      Wk, bk.reshape(1, DIM), Wv, bv.reshape(1, DIM), Wo, bo.reshape(1, 1),
      W1, b1.reshape(1, HID), g1.reshape(1, HID), be1.reshape(1, HID),
      W2, b2.reshape(1, 4))
    return out
